# baseline (speedup 1.0000x reference)
import functools

import jax
import jax.numpy as jnp
from jax import lax
from jax.experimental import pallas as pl
from jax.experimental.pallas import tpu as pltpu
from jax.experimental.pallas import tpu_sc as plsc

_NC = 2
_NS = 16
_NW = _NC * _NS
_K = 128
_LANES = 16


def _cdiv(a, b):
    return (a + b - 1) // b


def _mesh():
    return plsc.VectorSubcoreMesh(core_axis_name="c", subcore_axis_name="s")


@functools.lru_cache(maxsize=None)
def _make_hist(c1, c2, np_):
    @functools.partial(
        pl.kernel,
        out_type=jax.ShapeDtypeStruct((_NW, 2, np_), jnp.float32),
        mesh=_mesh(),
        compiler_params=pltpu.CompilerParams(needs_layout_passes=False),
        scratch_types=[
            pltpu.VMEM((np_,), jnp.float32),
            pltpu.VMEM((np_,), jnp.float32),
            pltpu.VMEM((c1, _K), jnp.int32),
            pltpu.VMEM((c2, _K), jnp.int32),
        ],
    )
    def hist(row1_hbm, row2_hbm, out_hbm, h1, h2, i1, i2):
        wid = lax.axis_index("c") * _NS + lax.axis_index("s")
        zeros = jnp.zeros((_LANES,), jnp.float32)
        ones = jnp.ones((_LANES,), jnp.float32)
        pltpu.sync_copy(row1_hbm.at[wid], i1)
        pltpu.sync_copy(row2_hbm.at[wid], i2)

        def zero_body(i, carry):
            h1[pl.ds(i * _LANES, _LANES)] = zeros
            h2[pl.ds(i * _LANES, _LANES)] = zeros
            return carry

        lax.fori_loop(0, np_ // _LANES, zero_body, 0)

        def count1(j, carry):
            for k in range(_K // _LANES):
                plsc.addupdate_scatter(h1, [i1[j, pl.ds(k * _LANES, _LANES)]], ones)
            return carry

        lax.fori_loop(0, c1, count1, 0)

        def count2(j, carry):
            for k in range(_K // _LANES):
                plsc.addupdate_scatter(h2, [i2[j, pl.ds(k * _LANES, _LANES)]], ones)
            return carry

        lax.fori_loop(0, c2, count2, 0)
        pltpu.sync_copy(h1, out_hbm.at[wid, 0])
        pltpu.sync_copy(h2, out_hbm.at[wid, 1])

    return hist


@functools.lru_cache(maxsize=None)
def _make_prop(c, np_, dh, g, k):
    rows_per_tile = np_ // _NS

    @functools.partial(
        pl.kernel,
        out_type=jax.ShapeDtypeStruct((_NC, np_, dh), jnp.float32),
        mesh=_mesh(),
        compiler_params=pltpu.CompilerParams(needs_layout_passes=False,
                                             use_tc_tiling_on_sc=False),
        scratch_types=[
            pltpu.VMEM((c, k), jnp.int32),
            pltpu.VMEM((c, k), jnp.int32),
            pltpu.VMEM((g * k, dh), jnp.float32),
            pltpu.VMEM_SHARED((np_, dh), jnp.float32),
            [pltpu.SemaphoreType.DMA] * g,
        ],
    )
    def prop(hs_hbm, row_hbm, col_hbm, out_hbm, rows_v, cols_v, buf0,
             acc, gsems):
        cid = lax.axis_index("c")
        sid = lax.axis_index("s")
        base = sid * rows_per_tile
        table = hs_hbm.at[cid]
        pltpu.sync_copy(table.at[pl.ds(base, rows_per_tile)],
                        acc.at[pl.ds(base, rows_per_tile)])
        pltpu.sync_copy(row_hbm.at[sid], rows_v)
        pltpu.sync_copy(col_hbm.at[sid], cols_v)
        plsc.subcore_barrier()

        bufs = [buf0.at[pl.ds(i * k, k)] for i in range(g)]
        for i in range(g):
            pltpu.async_copy(table.at[rows_v.at[i]], bufs[i], gsems[i])

        def body(t, carry):
            j0 = g * t
            for i in range(g):
                pltpu.make_async_copy(table.at[pl.ds(0, k)], bufs[i],
                                      gsems[i]).wait()
                pltpu.sync_copy(bufs[i], acc.at[cols_v.at[j0 + i]], add=True)

                @pl.when(j0 + i + g < c)
                def _():
                    pltpu.async_copy(table.at[rows_v.at[j0 + i + g]], bufs[i],
                                     gsems[i])

            return carry

        lax.fori_loop(0, c // g, body, 0)
        plsc.subcore_barrier()
        pltpu.sync_copy(acc.at[pl.ds(base, rows_per_tile)],
                        out_hbm.at[cid, pl.ds(base, rows_per_tile)])

    return prop


def _prep_kernel(x_ref, w_ref, h_ref, hs_ref, dis_ref):
    dis = lax.rsqrt(jnp.sum(h_ref[...], axis=0) + 1.0)
    dis_ref[...] = dis
    h = lax.dot_general(x_ref[...], w_ref[...], (((1,), (1,)), ((), ())),
                        preferred_element_type=jnp.float32)
    h = jnp.maximum(h, 0.0) * dis[0][:, None]
    dh = h.shape[1] // 2
    hs_ref[0] = h[:, :dh]
    hs_ref[1] = h[:, dh:]


def _mid_kernel(a_ref, d1, d2, b, o_ref):
    p = jnp.concatenate([a_ref[0], a_ref[1]], axis=1)
    h = d2[...] * jnp.maximum(d1[...] * p + b[...], 0.0)
    dh = h.shape[1] // 2
    o_ref[0] = h[:, :dh]
    o_ref[1] = h[:, dh:]


def _final_kernel(a_ref, d2, b, o_ref):
    p = jnp.concatenate([a_ref[0], a_ref[1]], axis=1)
    o_ref[...] = d2[...] * p + b[...]


def kernel(x, edge_index, edge_attr, W, bias1, bias2):
    n, d = x.shape
    dh = d // 2
    np_ = _cdiv(n + 1, 128) * 128
    e1 = n - 1

    ei = edge_index.astype(jnp.int32)
    r1, c1 = ei[0, :e1], ei[1, :e1]
    r2, c2 = ei[0, e1:], ei[1, e1:]
    row1 = jnp.concatenate([r1, c1])
    col1 = jnp.concatenate([c1, r1])
    row2 = jnp.concatenate([r2, c2])
    col2 = jnp.concatenate([c2, r2])

    def padded(a, nt, cn, k):
        full = nt * cn * k
        a = jnp.concatenate([a, jnp.full((full - a.shape[0],), n, jnp.int32)])
        return a.reshape(nt, cn, k)

    c1h = _cdiv(row1.shape[0], _NW * _K)
    c2h = _cdiv(row2.shape[0], _NW * _K)
    hists = _make_hist(c1h, c2h, np_)(padded(row1, _NW, c1h, _K),
                                      padded(row2, _NW, c2h, _K))

    g1, k1, g2, k2 = 2, 64, 2, 96
    c1p = g1 * _cdiv(row1.shape[0], _NS * k1 * g1)
    c2p = g2 * _cdiv(row2.shape[0], _NS * k2 * g2)
    row1q, col1q = padded(row1, _NS, c1p, k1), padded(col1, _NS, c1p, k1)
    row2q, col2q = padded(row2, _NS, c2p, k2), padded(col2, _NS, c2p, k2)

    x_p = jnp.zeros((np_, d), x.dtype).at[:n].set(x)
    hs1, dis = pl.pallas_call(
        _prep_kernel,
        out_shape=(jax.ShapeDtypeStruct((2, np_, dh), jnp.float32),
                   jax.ShapeDtypeStruct((2, np_), jnp.float32)),
    )(x_p, W, hists)
    dis1 = dis[0][:, None]
    dis2 = dis[1][:, None]

    br = np_ // 4
    grid = (np_ // br,)
    blk = pl.BlockSpec((br, d), lambda i: (i, 0))
    sblk = pl.BlockSpec((2, br, dh), lambda i: (0, i, 0))
    colb = pl.BlockSpec((br, 1), lambda i: (i, 0))
    wblk = pl.BlockSpec((d, d), lambda i: (0, 0))
    bblk = pl.BlockSpec((1, d), lambda i: (0, 0))
    sshape = jax.ShapeDtypeStruct((2, np_, dh), jnp.float32)

    acc1 = _make_prop(c1p, np_, dh, g1, k1)(hs1, row1q, col1q)

    hs2 = pl.pallas_call(
        _mid_kernel, grid=grid,
        in_specs=[sblk, colb, colb, bblk], out_specs=sblk, out_shape=sshape,
    )(acc1, dis1, dis2, bias1[None, :])

    acc2 = _make_prop(c2p, np_, dh, g2, k2)(hs2, row2q, col2q)

    outp = pl.pallas_call(
        _final_kernel, grid=grid,
        in_specs=[sblk, colb, bblk], out_specs=blk,
        out_shape=jax.ShapeDtypeStruct((np_, d), jnp.float32),
    )(acc2, dis2, bias2[None, :])

    return outp[:n]


# speedup vs baseline: 1.1078x; 1.1078x over previous
import functools

import jax
import jax.numpy as jnp
from jax import lax
from jax.experimental import pallas as pl
from jax.experimental.pallas import tpu as pltpu
from jax.experimental.pallas import tpu_sc as plsc

_NC = 2
_NS = 16
_NW = _NC * _NS
_K = 128
_LANES = 16


def _cdiv(a, b):
    return (a + b - 1) // b


def _mesh():
    return plsc.VectorSubcoreMesh(core_axis_name="c", subcore_axis_name="s")


@functools.lru_cache(maxsize=None)
def _make_hist(c1, c2, np_):
    @functools.partial(
        pl.kernel,
        out_type=jax.ShapeDtypeStruct((_NW, 2, np_), jnp.float32),
        mesh=_mesh(),
        compiler_params=pltpu.CompilerParams(needs_layout_passes=False),
        scratch_types=[
            pltpu.VMEM((np_,), jnp.float32),
            pltpu.VMEM((np_,), jnp.float32),
            pltpu.VMEM((c1, _K), jnp.int32),
            pltpu.VMEM((c2, _K), jnp.int32),
        ],
    )
    def hist(row1_hbm, row2_hbm, out_hbm, h1, h2, i1, i2):
        wid = lax.axis_index("c") * _NS + lax.axis_index("s")
        zeros = jnp.zeros((_LANES,), jnp.float32)
        ones = jnp.ones((_LANES,), jnp.float32)
        pltpu.sync_copy(row1_hbm.at[wid], i1)
        pltpu.sync_copy(row2_hbm.at[wid], i2)

        def zero_body(i, carry):
            h1[pl.ds(i * _LANES, _LANES)] = zeros
            h2[pl.ds(i * _LANES, _LANES)] = zeros
            return carry

        lax.fori_loop(0, np_ // _LANES, zero_body, 0)

        def count1(j, carry):
            for k in range(_K // _LANES):
                plsc.addupdate_scatter(h1, [i1[j, pl.ds(k * _LANES, _LANES)]], ones)
            return carry

        lax.fori_loop(0, c1, count1, 0)

        def count2(j, carry):
            for k in range(_K // _LANES):
                plsc.addupdate_scatter(h2, [i2[j, pl.ds(k * _LANES, _LANES)]], ones)
            return carry

        lax.fori_loop(0, c2, count2, 0)
        pltpu.sync_copy(h1, out_hbm.at[wid, 0])
        pltpu.sync_copy(h2, out_hbm.at[wid, 1])

    return hist


@functools.lru_cache(maxsize=None)
def _make_prop(c, np_, dh, g, k):
    rows_per_tile = np_ // _NS

    @functools.partial(
        pl.kernel,
        out_type=jax.ShapeDtypeStruct((_NC, np_, dh), jnp.float32),
        mesh=_mesh(),
        compiler_params=pltpu.CompilerParams(needs_layout_passes=False,
                                             use_tc_tiling_on_sc=False),
        scratch_types=[
            pltpu.VMEM((c, k), jnp.int32),
            pltpu.VMEM((c, k), jnp.int32),
            pltpu.VMEM((g * k, dh), jnp.float32),
            pltpu.VMEM_SHARED((np_, dh), jnp.float32),
            [pltpu.SemaphoreType.DMA] * g,
        ],
    )
    def prop(hs_hbm, row_hbm, col_hbm, out_hbm, rows_v, cols_v, buf0,
             acc, gsems):
        cid = lax.axis_index("c")
        sid = lax.axis_index("s")
        base = sid * rows_per_tile
        table = hs_hbm.at[cid]
        pltpu.sync_copy(table.at[pl.ds(base, rows_per_tile)],
                        acc.at[pl.ds(base, rows_per_tile)])
        pltpu.sync_copy(row_hbm.at[sid], rows_v)
        pltpu.sync_copy(col_hbm.at[sid], cols_v)
        plsc.subcore_barrier()

        bufs = [buf0.at[pl.ds(i * k, k)] for i in range(g)]
        for i in range(g):
            pltpu.async_copy(table.at[rows_v.at[i]], bufs[i], gsems[i])

        def body(t, carry):
            j0 = g * t
            for i in range(g):
                pltpu.make_async_copy(table.at[pl.ds(0, k)], bufs[i],
                                      gsems[i]).wait()
                pltpu.sync_copy(bufs[i], acc.at[cols_v.at[j0 + i]], add=True)

                @pl.when(j0 + i + g < c)
                def _():
                    pltpu.async_copy(table.at[rows_v.at[j0 + i + g]], bufs[i],
                                     gsems[i])

            return carry

        lax.fori_loop(0, c // g, body, 0)
        plsc.subcore_barrier()
        pltpu.sync_copy(acc.at[pl.ds(base, rows_per_tile)],
                        out_hbm.at[cid, pl.ds(base, rows_per_tile)])

    return prop


def _prep_kernel(x_ref, w_ref, h_ref, hs_ref, dis_ref):
    dis = lax.rsqrt(jnp.sum(h_ref[...], axis=0) + 1.0)
    dis_ref[...] = dis
    h = lax.dot_general(x_ref[...], w_ref[...], (((1,), (1,)), ((), ())),
                        preferred_element_type=jnp.float32)
    h = jnp.maximum(h, 0.0) * dis[0][:, None]
    dh = h.shape[1] // 2
    hs_ref[0] = h[:, :dh]
    hs_ref[1] = h[:, dh:]


def _mid_kernel(a_ref, d1, d2, b, o_ref):
    p = jnp.concatenate([a_ref[0], a_ref[1]], axis=1)
    h = d2[...] * jnp.maximum(d1[...] * p + b[...], 0.0)
    dh = h.shape[1] // 2
    o_ref[0] = h[:, :dh]
    o_ref[1] = h[:, dh:]


def _final_kernel(a_ref, d2, b, o_ref):
    p = jnp.concatenate([a_ref[0], a_ref[1]], axis=1)
    o_ref[...] = d2[...] * p + b[...]


def kernel(x, edge_index, edge_attr, W, bias1, bias2):
    n, d = x.shape
    dh = d // 2
    np_ = _cdiv(n + 1, 128) * 128
    e1 = n - 1

    ei = edge_index.astype(jnp.int32)
    r1, c1 = ei[0, :e1], ei[1, :e1]
    r2, c2 = ei[0, e1:], ei[1, e1:]
    row1 = jnp.concatenate([r1, c1])
    col1 = jnp.concatenate([c1, r1])
    row2 = jnp.concatenate([r2, c2])
    col2 = jnp.concatenate([c2, r2])

    def padded(a, nt, cn, k):
        full = nt * cn * k
        a = jnp.concatenate([a, jnp.full((full - a.shape[0],), n, jnp.int32)])
        return a.reshape(nt, cn, k)

    c1h = _cdiv(row1.shape[0], _NW * _K)
    c2h = _cdiv(row2.shape[0], _NW * _K)
    hists = _make_hist(c1h, c2h, np_)(padded(row1, _NW, c1h, _K),
                                      padded(row2, _NW, c2h, _K))

    g1, k1, g2, k2 = 2, 64, 4, 48
    c1p = g1 * _cdiv(row1.shape[0], _NS * k1 * g1)
    c2p = g2 * _cdiv(row2.shape[0], _NS * k2 * g2)
    row1q, col1q = padded(row1, _NS, c1p, k1), padded(col1, _NS, c1p, k1)
    row2q, col2q = padded(row2, _NS, c2p, k2), padded(col2, _NS, c2p, k2)

    x_p = jnp.zeros((np_, d), x.dtype).at[:n].set(x)
    hs1, dis = pl.pallas_call(
        _prep_kernel,
        out_shape=(jax.ShapeDtypeStruct((2, np_, dh), jnp.float32),
                   jax.ShapeDtypeStruct((2, np_), jnp.float32)),
    )(x_p, W, hists)
    dis1 = dis[0][:, None]
    dis2 = dis[1][:, None]

    br = np_ // 4
    grid = (np_ // br,)
    blk = pl.BlockSpec((br, d), lambda i: (i, 0))
    sblk = pl.BlockSpec((2, br, dh), lambda i: (0, i, 0))
    colb = pl.BlockSpec((br, 1), lambda i: (i, 0))
    wblk = pl.BlockSpec((d, d), lambda i: (0, 0))
    bblk = pl.BlockSpec((1, d), lambda i: (0, 0))
    sshape = jax.ShapeDtypeStruct((2, np_, dh), jnp.float32)

    acc1 = _make_prop(c1p, np_, dh, g1, k1)(hs1, row1q, col1q)

    hs2 = pl.pallas_call(
        _mid_kernel, grid=grid,
        in_specs=[sblk, colb, colb, bblk], out_specs=sblk, out_shape=sshape,
    )(acc1, dis1, dis2, bias1[None, :])

    acc2 = _make_prop(c2p, np_, dh, g2, k2)(hs2, row2q, col2q)

    outp = pl.pallas_call(
        _final_kernel, grid=grid,
        in_specs=[sblk, colb, bblk], out_specs=blk,
        out_shape=jax.ShapeDtypeStruct((np_, d), jnp.float32),
    )(acc2, dis2, bias2[None, :])

    return outp[:n]


# speedup vs baseline: 1.1282x; 1.0184x over previous
import functools

import jax
import jax.numpy as jnp
from jax import lax
from jax.experimental import pallas as pl
from jax.experimental.pallas import tpu as pltpu
from jax.experimental.pallas import tpu_sc as plsc

_NC = 2
_NS = 16
_NW = _NC * _NS
_K = 128
_LANES = 16


def _cdiv(a, b):
    return (a + b - 1) // b


def _mesh():
    return plsc.VectorSubcoreMesh(core_axis_name="c", subcore_axis_name="s")


@functools.lru_cache(maxsize=None)
def _make_hist(c1, c2, np_):
    @functools.partial(
        pl.kernel,
        out_type=jax.ShapeDtypeStruct((_NW, 2, np_), jnp.float32),
        mesh=_mesh(),
        compiler_params=pltpu.CompilerParams(needs_layout_passes=False),
        scratch_types=[
            pltpu.VMEM((np_,), jnp.float32),
            pltpu.VMEM((np_,), jnp.float32),
            pltpu.VMEM((c1, _K), jnp.int32),
            pltpu.VMEM((c2, _K), jnp.int32),
        ],
    )
    def hist(row1_hbm, row2_hbm, out_hbm, h1, h2, i1, i2):
        wid = lax.axis_index("c") * _NS + lax.axis_index("s")
        zeros = jnp.zeros((_LANES,), jnp.float32)
        ones = jnp.ones((_LANES,), jnp.float32)
        pltpu.sync_copy(row1_hbm.at[wid], i1)
        pltpu.sync_copy(row2_hbm.at[wid], i2)

        def zero_body(i, carry):
            h1[pl.ds(i * _LANES, _LANES)] = zeros
            h2[pl.ds(i * _LANES, _LANES)] = zeros
            return carry

        lax.fori_loop(0, np_ // _LANES, zero_body, 0)

        def count1(j, carry):
            for k in range(_K // _LANES):
                plsc.addupdate_scatter(h1, [i1[j, pl.ds(k * _LANES, _LANES)]], ones)
            return carry

        lax.fori_loop(0, c1, count1, 0)

        def count2(j, carry):
            for k in range(_K // _LANES):
                plsc.addupdate_scatter(h2, [i2[j, pl.ds(k * _LANES, _LANES)]], ones)
            return carry

        lax.fori_loop(0, c2, count2, 0)
        pltpu.sync_copy(h1, out_hbm.at[wid, 0])
        pltpu.sync_copy(h2, out_hbm.at[wid, 1])

    return hist


@functools.lru_cache(maxsize=None)
def _make_prop(c, np_, dh, g, k):
    rows_per_tile = np_ // _NS

    @functools.partial(
        pl.kernel,
        out_type=jax.ShapeDtypeStruct((_NC, np_, dh), jnp.float32),
        mesh=_mesh(),
        compiler_params=pltpu.CompilerParams(needs_layout_passes=False,
                                             use_tc_tiling_on_sc=False),
        scratch_types=[
            pltpu.VMEM((c, k), jnp.int32),
            pltpu.VMEM((c, k), jnp.int32),
            pltpu.VMEM((g * k, dh), jnp.float32),
            pltpu.VMEM_SHARED((np_, dh), jnp.float32),
            [pltpu.SemaphoreType.DMA] * g,
        ],
    )
    def prop(hs_hbm, row_hbm, col_hbm, out_hbm, rows_v, cols_v, buf0,
             acc, gsems):
        cid = lax.axis_index("c")
        sid = lax.axis_index("s")
        base = sid * rows_per_tile
        table = hs_hbm.at[cid]
        pltpu.sync_copy(table.at[pl.ds(base, rows_per_tile)],
                        acc.at[pl.ds(base, rows_per_tile)])
        pltpu.sync_copy(row_hbm.at[sid], rows_v)
        pltpu.sync_copy(col_hbm.at[sid], cols_v)
        plsc.subcore_barrier()

        bufs = [buf0.at[pl.ds(i * k, k)] for i in range(g)]
        for i in range(g):
            pltpu.async_copy(table.at[rows_v.at[i]], bufs[i], gsems[i])

        def body(t, carry):
            j0 = g * t
            for i in range(g):
                pltpu.make_async_copy(table.at[pl.ds(0, k)], bufs[i],
                                      gsems[i]).wait()
                pltpu.sync_copy(bufs[i], acc.at[cols_v.at[j0 + i]], add=True)

                @pl.when(j0 + i + g < c)
                def _():
                    pltpu.async_copy(table.at[rows_v.at[j0 + i + g]], bufs[i],
                                     gsems[i])

            return carry

        lax.fori_loop(0, c // g, body, 0)
        plsc.subcore_barrier()
        pltpu.sync_copy(acc.at[pl.ds(base, rows_per_tile)],
                        out_hbm.at[cid, pl.ds(base, rows_per_tile)])

    return prop


def _prep_kernel(x_ref, w_ref, h_ref, hs_ref, dis_ref):
    dis = lax.rsqrt(jnp.sum(h_ref[...], axis=0) + 1.0)
    dis_ref[...] = dis
    h = lax.dot_general(x_ref[...], w_ref[...], (((1,), (1,)), ((), ())),
                        preferred_element_type=jnp.float32)
    h = jnp.maximum(h, 0.0) * dis[0][:, None]
    dh = h.shape[1] // 2
    hs_ref[0] = h[:, :dh]
    hs_ref[1] = h[:, dh:]


def _mid_kernel(a_ref, d1, d2, b, o_ref):
    p = jnp.concatenate([a_ref[0], a_ref[1]], axis=1)
    h = d2[...] * jnp.maximum(d1[...] * p + b[...], 0.0)
    dh = h.shape[1] // 2
    o_ref[0] = h[:, :dh]
    o_ref[1] = h[:, dh:]


def _final_kernel(a_ref, d2, b, o_ref):
    p = jnp.concatenate([a_ref[0], a_ref[1]], axis=1)
    o_ref[...] = d2[...] * p + b[...]


def kernel(x, edge_index, edge_attr, W, bias1, bias2):
    n, d = x.shape
    dh = d // 2
    np_ = _cdiv(n + 1, 128) * 128
    e1 = n - 1

    ei = edge_index.astype(jnp.int32)
    r1, c1 = ei[0, :e1], ei[1, :e1]
    r2, c2 = ei[0, e1:], ei[1, e1:]
    row1 = jnp.concatenate([r1, c1])
    col1 = jnp.concatenate([c1, r1])
    row2 = jnp.concatenate([r2, c2])
    col2 = jnp.concatenate([c2, r2])

    def padded(a, nt, cn, k):
        full = nt * cn * k
        a = jnp.concatenate([a, jnp.full((full - a.shape[0],), n, jnp.int32)])
        return a.reshape(nt, cn, k)

    c1h = _cdiv(row1.shape[0], _NW * _K)
    c2h = _cdiv(row2.shape[0], _NW * _K)
    hists = _make_hist(c1h, c2h, np_)(padded(row1, _NW, c1h, _K),
                                      padded(row2, _NW, c2h, _K))

    g1, k1, g2, k2 = 4, 32, 6, 32
    c1p = g1 * _cdiv(row1.shape[0], _NS * k1 * g1)
    c2p = g2 * _cdiv(row2.shape[0], _NS * k2 * g2)
    row1q, col1q = padded(row1, _NS, c1p, k1), padded(col1, _NS, c1p, k1)
    row2q, col2q = padded(row2, _NS, c2p, k2), padded(col2, _NS, c2p, k2)

    x_p = jnp.zeros((np_, d), x.dtype).at[:n].set(x)
    hs1, dis = pl.pallas_call(
        _prep_kernel,
        out_shape=(jax.ShapeDtypeStruct((2, np_, dh), jnp.float32),
                   jax.ShapeDtypeStruct((2, np_), jnp.float32)),
    )(x_p, W, hists)
    dis1 = dis[0][:, None]
    dis2 = dis[1][:, None]

    br = np_ // 4
    grid = (np_ // br,)
    blk = pl.BlockSpec((br, d), lambda i: (i, 0))
    sblk = pl.BlockSpec((2, br, dh), lambda i: (0, i, 0))
    colb = pl.BlockSpec((br, 1), lambda i: (i, 0))
    wblk = pl.BlockSpec((d, d), lambda i: (0, 0))
    bblk = pl.BlockSpec((1, d), lambda i: (0, 0))
    sshape = jax.ShapeDtypeStruct((2, np_, dh), jnp.float32)

    acc1 = _make_prop(c1p, np_, dh, g1, k1)(hs1, row1q, col1q)

    hs2 = pl.pallas_call(
        _mid_kernel, grid=grid,
        in_specs=[sblk, colb, colb, bblk], out_specs=sblk, out_shape=sshape,
    )(acc1, dis1, dis2, bias1[None, :])

    acc2 = _make_prop(c2p, np_, dh, g2, k2)(hs2, row2q, col2q)

    outp = pl.pallas_call(
        _final_kernel, grid=grid,
        in_specs=[sblk, colb, bblk], out_specs=blk,
        out_shape=jax.ShapeDtypeStruct((np_, d), jnp.float32),
    )(acc2, dis2, bias2[None, :])

    return outp[:n]
